# baseline (speedup 1.0000x reference)
import functools

import jax
import jax.numpy as jnp
from jax import lax
from jax.experimental import pallas as pl
from jax.experimental.pallas import tpu as pltpu
from jax.experimental.pallas import tpu_sc as plsc

N = 10000
E = 320000
D = 128
F = 16

NC = 2
NS = 16
NW = NC * NS

EPT = E // NW
G = 64
NFULL = EPT // G
TAIL = EPT - NFULL * G
NPAD = 10240
ROWS_PER_SUB = NPAD // NS
LANES = 16


def _scale_body(f_ref, w_ref, o_ref):
    o_ref[...] = jnp.sum(f_ref[...], axis=-1) * jnp.sum(w_ref[...], axis=-1)


def _compute_scale(edge_filter, weight):
    BR = 512
    return pl.pallas_call(
        _scale_body,
        grid=(E // BR,),
        in_specs=[
            pl.BlockSpec((BR, F), lambda i: (i, 0)),
            pl.BlockSpec((BR, F), lambda i: (i, 0)),
        ],
        out_specs=pl.BlockSpec((BR,), lambda i: (i,)),
        out_shape=jax.ShapeDtypeStruct((E,), jnp.float32),
    )(edge_filter, weight)


def _add_body(p_ref, o_ref):
    o_ref[...] = p_ref[0] + p_ref[1]


def _combine(partial):
    BR = 2000
    return pl.pallas_call(
        _add_body,
        grid=(N // BR,),
        in_specs=[pl.BlockSpec((NC, BR, D), lambda i: (0, i, 0))],
        out_specs=pl.BlockSpec((BR, D), lambda i: (i, 0)),
        out_shape=jax.ShapeDtypeStruct((N, D), jnp.float32),
    )(partial)


def _sc_main(x, scale, src, dst):
    mesh = plsc.VectorSubcoreMesh(core_axis_name="c", subcore_axis_name="s")

    @functools.partial(
        pl.kernel,
        out_type=jax.ShapeDtypeStruct((NC, NPAD, D), jnp.float32),
        mesh=mesh,
        scratch_types=[
            pltpu.VMEM((EPT,), jnp.int32),
            pltpu.VMEM((EPT,), jnp.int32),
            pltpu.VMEM((EPT,), jnp.float32),
            pltpu.VMEM((2, G, D), jnp.float32),
            pltpu.VMEM((2, G), jnp.int32),
            pltpu.VMEM((TAIL, D), jnp.float32),
            pltpu.VMEM((TAIL,), jnp.int32),
            pltpu.VMEM_SHARED((NPAD, D), jnp.float32),
            pltpu.SemaphoreType.DMA,
            pltpu.SemaphoreType.DMA,
        ],
    )
    def body(x_hbm, scale_hbm, src_hbm, dst_hbm, out_hbm,
             src_v, dst_v, scale_v, rows_v, dstg_v, trows_v, tdst_v, acc,
             sem0, sem1):
        cid = lax.axis_index("c")
        sid = lax.axis_index("s")
        wid = cid * NS + sid
        base = wid * EPT
        sems = [sem0, sem1]

        pltpu.sync_copy(src_hbm.at[pl.ds(base, EPT)], src_v)
        pltpu.sync_copy(dst_hbm.at[pl.ds(base, EPT)], dst_v)
        pltpu.sync_copy(scale_hbm.at[pl.ds(base, EPT)], scale_v)

        zero = jnp.zeros((LANES,), jnp.float32)

        def zrow(i, carry):
            for b in range(2):
                for q in range(D // LANES):
                    rows_v[b, i, pl.ds(q * LANES, LANES)] = zero
            return carry

        lax.fori_loop(0, G, zrow, 0)
        for k in range(ROWS_PER_SUB // G):
            pltpu.sync_copy(
                rows_v.at[k % 2],
                acc.at[pl.ds(sid * ROWS_PER_SUB + k * G, G)])
        plsc.subcore_barrier()

        def issue_gather(j, b):
            return pltpu.async_copy(
                x_hbm.at[src_v.at[pl.ds(j * G, G)]], rows_v.at[b], sems[b])

        def stage_dst(j, b):
            for g in range(G // LANES):
                dstg_v[b, pl.ds(g * LANES, LANES)] = (
                    dst_v[pl.ds(j * G + g * LANES, LANES)])

        for b in range(2):
            issue_gather(b, b)
            stage_dst(b, b)

        def pair_body(p, carry):
            j0 = p * 2
            for b in range(2):
                j = j0 + b
                pltpu.make_async_copy(
                    x_hbm.at[src_v.at[pl.ds(0, G)]], rows_v.at[b],
                    sems[b]).wait()
                for g in range(G // LANES):
                    s16 = scale_v[pl.ds(j * G + g * LANES, LANES)]
                    for t in range(LANES):
                        e = g * LANES + t
                        s = s16[t]
                        for q in range(D // LANES):
                            sl = pl.ds(q * LANES, LANES)
                            rows_v[b, e, sl] = rows_v[b, e, sl] * s
                pltpu.sync_copy(rows_v.at[b], acc.at[dstg_v.at[b]], add=True)

                @pl.when(j + 2 < NFULL)
                def _():
                    issue_gather(j + 2, b)
                    stage_dst(j + 2, b)
            return carry

        lax.fori_loop(0, NFULL // 2, pair_body, 0)

        tdst_v[...] = dst_v[pl.ds(NFULL * G, TAIL)]
        pltpu.async_copy(
            x_hbm.at[src_v.at[pl.ds(NFULL * G, TAIL)]], trows_v, sem0).wait()
        s16 = scale_v[pl.ds(NFULL * G, LANES)]
        for t in range(TAIL):
            s = s16[t]
            for q in range(D // LANES):
                sl = pl.ds(q * LANES, LANES)
                trows_v[t, sl] = trows_v[t, sl] * s
        pltpu.sync_copy(trows_v, acc.at[tdst_v], add=True)

        plsc.subcore_barrier()

        for k in range(ROWS_PER_SUB // 128):
            r0 = sid * ROWS_PER_SUB + k * 128
            pltpu.sync_copy(acc.at[pl.ds(r0, 128)],
                            out_hbm.at[cid, pl.ds(r0, 128)])

    return body(x, scale, src, dst)


def kernel(x, edge_filter, weight, edge_src, edge_dst):
    scale = _compute_scale(edge_filter, weight)
    partial = _sc_main(x, scale,
                       edge_src.astype(jnp.int32), edge_dst.astype(jnp.int32))
    return _combine(partial)


# speedup vs baseline: 2.4224x; 2.4224x over previous
import functools

import jax
import jax.numpy as jnp
from jax import lax
from jax.experimental import pallas as pl
from jax.experimental.pallas import tpu as pltpu
from jax.experimental.pallas import tpu_sc as plsc

N = 10000
E = 320000
D = 128
F = 16

NC = 2
NS = 16
NW = NC * NS

EPT = E // NW
G = 64
NFULL = EPT // G
TAIL = EPT - NFULL * G
NPAD = 10240
ROWS_PER_SUB = NPAD // NS
LANES = 16


def _scale_body(f_ref, w_ref, o_ref):
    o_ref[...] = jnp.sum(f_ref[...], axis=-1) * jnp.sum(w_ref[...], axis=-1)


def _compute_scale(edge_filter, weight):
    BR = 200
    return pl.pallas_call(
        _scale_body,
        grid=(4000 // BR,),
        in_specs=[
            pl.BlockSpec((BR, 80, F), lambda i: (i, 0, 0)),
            pl.BlockSpec((BR, 80, F), lambda i: (i, 0, 0)),
        ],
        out_specs=pl.BlockSpec((BR, 80), lambda i: (i, 0)),
        out_shape=jax.ShapeDtypeStruct((4000, 80), jnp.float32),
    )(edge_filter.reshape(4000, 80, F), weight.reshape(4000, 80, F))


def _add_body(p_ref, o_ref):
    o_ref[...] = p_ref[0] + p_ref[1]


def _combine(partial):
    BR = 2000
    return pl.pallas_call(
        _add_body,
        grid=(N // BR,),
        in_specs=[pl.BlockSpec((NC, BR, D), lambda i: (0, i, 0))],
        out_specs=pl.BlockSpec((BR, D), lambda i: (i, 0)),
        out_shape=jax.ShapeDtypeStruct((N, D), jnp.float32),
    )(partial)


def _sc_main(x, scale, src, dst):
    mesh = plsc.VectorSubcoreMesh(core_axis_name="c", subcore_axis_name="s")

    @functools.partial(
        pl.kernel,
        out_type=jax.ShapeDtypeStruct((NC, NPAD, D), jnp.float32),
        mesh=mesh,
        scratch_types=[
            pltpu.VMEM((EPT,), jnp.int32),
            pltpu.VMEM((EPT,), jnp.int32),
            pltpu.VMEM((EPT,), jnp.float32),
            pltpu.VMEM((2, G, D), jnp.float32),
            pltpu.VMEM((2, G), jnp.int32),
            pltpu.VMEM((TAIL, D), jnp.float32),
            pltpu.VMEM((TAIL,), jnp.int32),
            pltpu.VMEM_SHARED((NPAD, D), jnp.float32),
            pltpu.SemaphoreType.DMA,
            pltpu.SemaphoreType.DMA,
        ],
    )
    def body(x_hbm, scale_hbm, src_hbm, dst_hbm, out_hbm,
             src_v, dst_v, scale_v, rows_v, dstg_v, trows_v, tdst_v, acc,
             sem0, sem1):
        cid = lax.axis_index("c")
        sid = lax.axis_index("s")
        wid = cid * NS + sid
        base = wid * EPT
        sems = [sem0, sem1]

        pltpu.sync_copy(src_hbm.at[pl.ds(base, EPT)], src_v)
        pltpu.sync_copy(dst_hbm.at[pl.ds(base, EPT)], dst_v)
        pltpu.sync_copy(scale_hbm.at[pl.ds(base, EPT)], scale_v)

        zero = jnp.zeros((LANES,), jnp.float32)

        def zrow(i, carry):
            for b in range(2):
                for q in range(D // LANES):
                    rows_v[b, i, pl.ds(q * LANES, LANES)] = zero
            return carry

        lax.fori_loop(0, G, zrow, 0)
        for k in range(ROWS_PER_SUB // G):
            pltpu.sync_copy(
                rows_v.at[k % 2],
                acc.at[pl.ds(sid * ROWS_PER_SUB + k * G, G)])
        plsc.subcore_barrier()

        def issue_gather(j, b):
            return pltpu.async_copy(
                x_hbm.at[src_v.at[pl.ds(j * G, G)]], rows_v.at[b], sems[b])

        def stage_dst(j, b):
            for g in range(G // LANES):
                dstg_v[b, pl.ds(g * LANES, LANES)] = (
                    dst_v[pl.ds(j * G + g * LANES, LANES)])

        for b in range(2):
            issue_gather(b, b)
            stage_dst(b, b)

        def pair_body(p, carry):
            j0 = p * 2
            for b in range(2):
                j = j0 + b
                pltpu.make_async_copy(
                    x_hbm.at[src_v.at[pl.ds(0, G)]], rows_v.at[b],
                    sems[b]).wait()
                for g in range(G // LANES):
                    s16 = scale_v[pl.ds(j * G + g * LANES, LANES)]
                    for t in range(LANES):
                        e = g * LANES + t
                        s = s16[t]
                        for q in range(D // LANES):
                            sl = pl.ds(q * LANES, LANES)
                            rows_v[b, e, sl] = rows_v[b, e, sl] * s
                pltpu.sync_copy(rows_v.at[b], acc.at[dstg_v.at[b]], add=True)

                @pl.when(j + 2 < NFULL)
                def _():
                    issue_gather(j + 2, b)
                    stage_dst(j + 2, b)
            return carry

        lax.fori_loop(0, NFULL // 2, pair_body, 0)

        tdst_v[...] = dst_v[pl.ds(NFULL * G, TAIL)]
        pltpu.async_copy(
            x_hbm.at[src_v.at[pl.ds(NFULL * G, TAIL)]], trows_v, sem0).wait()
        s16 = scale_v[pl.ds(NFULL * G, LANES)]
        for t in range(TAIL):
            s = s16[t]
            for q in range(D // LANES):
                sl = pl.ds(q * LANES, LANES)
                trows_v[t, sl] = trows_v[t, sl] * s
        pltpu.sync_copy(trows_v, acc.at[tdst_v], add=True)

        plsc.subcore_barrier()

        for k in range(ROWS_PER_SUB // 128):
            r0 = sid * ROWS_PER_SUB + k * 128
            pltpu.sync_copy(acc.at[pl.ds(r0, 128)],
                            out_hbm.at[cid, pl.ds(r0, 128)])

    return body(x, scale, src, dst)


def kernel(x, edge_filter, weight, edge_src, edge_dst):
    scale = _compute_scale(edge_filter, weight).reshape(E)
    partial = _sc_main(x, scale,
                       edge_src.astype(jnp.int32), edge_dst.astype(jnp.int32))
    return _combine(partial)


# speedup vs baseline: 4.1558x; 1.7155x over previous
import functools

import jax
import jax.numpy as jnp
from jax import lax
from jax.experimental import pallas as pl
from jax.experimental.pallas import tpu as pltpu
from jax.experimental.pallas import tpu_sc as plsc

N = 10000
E = 320000
D = 128
F = 16

NC = 2
NS = 16
NW = NC * NS

EPT = E // NW
G = 64
NFULL = EPT // G
TAIL = EPT - NFULL * G
NPAD = 10240
ROWS_PER_SUB = NPAD // NS
LANES = 16


BX = 16000


def _scale_body(f_ref, w_ref, o_ref):
    i = pl.program_id(0)
    s = jnp.sum(f_ref[...], axis=0) * jnp.sum(w_ref[...], axis=0)
    o_ref[pl.ds(i * BX, BX)] = s


def _compute_scale(edge_filter_t, weight_t):
    return pl.pallas_call(
        _scale_body,
        grid=(E // BX,),
        in_specs=[
            pl.BlockSpec((F, BX), lambda i: (0, i)),
            pl.BlockSpec((F, BX), lambda i: (0, i)),
        ],
        out_specs=pl.BlockSpec((E,), lambda i: (0,)),
        out_shape=jax.ShapeDtypeStruct((E,), jnp.float32),
    )(edge_filter_t, weight_t)


def _add_body(p_ref, o_ref):
    o_ref[...] = p_ref[0] + p_ref[1]


def _combine(partial):
    BR = 2000
    return pl.pallas_call(
        _add_body,
        grid=(N // BR,),
        in_specs=[pl.BlockSpec((NC, BR, D), lambda i: (0, i, 0))],
        out_specs=pl.BlockSpec((BR, D), lambda i: (i, 0)),
        out_shape=jax.ShapeDtypeStruct((N, D), jnp.float32),
    )(partial)


def _sc_main(x, scale, src, dst):
    mesh = plsc.VectorSubcoreMesh(core_axis_name="c", subcore_axis_name="s")

    @functools.partial(
        pl.kernel,
        out_type=jax.ShapeDtypeStruct((NC, NPAD, D), jnp.float32),
        mesh=mesh,
        scratch_types=[
            pltpu.VMEM((EPT,), jnp.int32),
            pltpu.VMEM((EPT,), jnp.int32),
            pltpu.VMEM((EPT,), jnp.float32),
            pltpu.VMEM((2, G, D), jnp.float32),
            pltpu.VMEM((2, G), jnp.int32),
            pltpu.VMEM((TAIL, D), jnp.float32),
            pltpu.VMEM((TAIL,), jnp.int32),
            pltpu.VMEM_SHARED((NPAD, D), jnp.float32),
            pltpu.SemaphoreType.DMA,
            pltpu.SemaphoreType.DMA,
        ],
    )
    def body(x_hbm, scale_hbm, src_hbm, dst_hbm, out_hbm,
             src_v, dst_v, scale_v, rows_v, dstg_v, trows_v, tdst_v, acc,
             sem0, sem1):
        cid = lax.axis_index("c")
        sid = lax.axis_index("s")
        wid = cid * NS + sid
        base = wid * EPT
        sems = [sem0, sem1]

        pltpu.sync_copy(src_hbm.at[pl.ds(base, EPT)], src_v)
        pltpu.sync_copy(dst_hbm.at[pl.ds(base, EPT)], dst_v)
        pltpu.sync_copy(scale_hbm.at[pl.ds(base, EPT)], scale_v)

        zero = jnp.zeros((LANES,), jnp.float32)

        def zrow(i, carry):
            for b in range(2):
                for q in range(D // LANES):
                    rows_v[b, i, pl.ds(q * LANES, LANES)] = zero
            return carry

        lax.fori_loop(0, G, zrow, 0)
        for k in range(ROWS_PER_SUB // G):
            pltpu.sync_copy(
                rows_v.at[k % 2],
                acc.at[pl.ds(sid * ROWS_PER_SUB + k * G, G)])
        plsc.subcore_barrier()

        def issue_gather(j, b):
            return pltpu.async_copy(
                x_hbm.at[src_v.at[pl.ds(j * G, G)]], rows_v.at[b], sems[b])

        def stage_dst(j, b):
            for g in range(G // LANES):
                dstg_v[b, pl.ds(g * LANES, LANES)] = (
                    dst_v[pl.ds(j * G + g * LANES, LANES)])

        for b in range(2):
            issue_gather(b, b)
            stage_dst(b, b)

        def pair_body(p, carry):
            j0 = p * 2
            for b in range(2):
                j = j0 + b
                pltpu.make_async_copy(
                    x_hbm.at[src_v.at[pl.ds(0, G)]], rows_v.at[b],
                    sems[b]).wait()
                for g in range(G // LANES):
                    s16 = scale_v[pl.ds(j * G + g * LANES, LANES)]
                    for t in range(LANES):
                        e = g * LANES + t
                        s = s16[t]
                        for q in range(D // LANES):
                            sl = pl.ds(q * LANES, LANES)
                            rows_v[b, e, sl] = rows_v[b, e, sl] * s
                pltpu.sync_copy(rows_v.at[b], acc.at[dstg_v.at[b]], add=True)

                @pl.when(j + 2 < NFULL)
                def _():
                    issue_gather(j + 2, b)
                    stage_dst(j + 2, b)
            return carry

        lax.fori_loop(0, NFULL // 2, pair_body, 0)

        tdst_v[...] = dst_v[pl.ds(NFULL * G, TAIL)]
        pltpu.async_copy(
            x_hbm.at[src_v.at[pl.ds(NFULL * G, TAIL)]], trows_v, sem0).wait()
        s16 = scale_v[pl.ds(NFULL * G, LANES)]
        for t in range(TAIL):
            s = s16[t]
            for q in range(D // LANES):
                sl = pl.ds(q * LANES, LANES)
                trows_v[t, sl] = trows_v[t, sl] * s
        pltpu.sync_copy(trows_v, acc.at[tdst_v], add=True)

        plsc.subcore_barrier()

        for k in range(ROWS_PER_SUB // 128):
            r0 = sid * ROWS_PER_SUB + k * 128
            pltpu.sync_copy(acc.at[pl.ds(r0, 128)],
                            out_hbm.at[cid, pl.ds(r0, 128)])

    return body(x, scale, src, dst)


def kernel(x, edge_filter, weight, edge_src, edge_dst):
    scale = _compute_scale(edge_filter.T, weight.T)
    partial = _sc_main(x, scale,
                       edge_src.astype(jnp.int32), edge_dst.astype(jnp.int32))
    return _combine(partial)
